# Initial kernel scaffold; baseline (speedup 1.0000x reference)
import jax
import jax.numpy as jnp
from jax.experimental import pallas as pl


def kernel(x, weights, expert_indices, gate_up, down, per_expert_scale):
    raise NotImplementedError("write your pallas kernel here")


# speedup vs baseline: 12.4688x; 12.4688x over previous
import functools

import jax
import jax.numpy as jnp
from jax.experimental import pallas as pl
from jax.experimental.pallas import tpu as pltpu


def _moe_kernel(idx_ref, w_ref, x_ref, scale_ref, gu_ref, dw_ref, out_ref, *, F):
    e = pl.program_id(0)

    @pl.when(e == 0)
    def _init():
        out_ref[...] = jnp.zeros_like(out_ref)

    mask = idx_ref[...] == e
    coeff = jnp.sum(jnp.where(mask, w_ref[...], 0.0), axis=1, keepdims=True)
    coeff = coeff * scale_ref[e]

    x = x_ref[...]
    gu = gu_ref[0]
    h = jnp.dot(x, gu, preferred_element_type=jnp.float32)
    gate = h[:, :F]
    up = h[:, F:]
    act = 0.5 * gate * (1.0 + jax.lax.erf(gate * 0.7071067811865476)) * up
    y = jnp.dot(act, dw_ref[0], preferred_element_type=jnp.float32)
    out_ref[...] += coeff * y


def kernel(x, weights, expert_indices, gate_up, down, per_expert_scale):
    B, L, D = x.shape
    K = weights.shape[-1]
    E, _, F2 = gate_up.shape
    F = F2 // 2
    N = B * L

    x_flat = x.reshape(N, D)
    w_flat = weights.reshape(N, K)
    idx_flat = expert_indices.reshape(N, K)

    out = pl.pallas_call(
        functools.partial(_moe_kernel, F=F),
        grid=(E,),
        in_specs=[
            pl.BlockSpec((N, K), lambda e: (0, 0)),
            pl.BlockSpec((N, K), lambda e: (0, 0)),
            pl.BlockSpec((N, D), lambda e: (0, 0)),
            pl.BlockSpec(memory_space=pltpu.SMEM),
            pl.BlockSpec((1, D, F2), lambda e: (e, 0, 0)),
            pl.BlockSpec((1, F, D), lambda e: (e, 0, 0)),
        ],
        out_specs=pl.BlockSpec((N, D), lambda e: (0, 0)),
        out_shape=jax.ShapeDtypeStruct((N, D), jnp.float32),
    )(idx_flat, w_flat, x_flat, per_expert_scale, gate_up, down)

    return out.reshape(B, L, D)


# speedup vs baseline: 12.4796x; 1.0009x over previous
import functools

import jax
import jax.numpy as jnp
from jax.experimental import pallas as pl
from jax.experimental.pallas import tpu as pltpu


def _moe_kernel(idx_ref, w_ref, x_ref, scale_ref, gu_ref, dw_ref, out_ref, *, F):
    e = pl.program_id(0)

    @pl.when(e == 0)
    def _init():
        out_ref[...] = jnp.zeros_like(out_ref)

    mask = idx_ref[...] == e
    coeff = jnp.sum(jnp.where(mask, w_ref[...], 0.0), axis=1, keepdims=True)
    coeff = coeff * scale_ref[e]

    x = x_ref[...].astype(jnp.bfloat16)
    gu = gu_ref[0].astype(jnp.bfloat16)
    h = jnp.dot(x, gu, preferred_element_type=jnp.float32)
    gate = h[:, :F]
    up = h[:, F:]
    act = 0.5 * gate * (1.0 + jax.lax.erf(gate * 0.7071067811865476)) * up
    y = jnp.dot(act.astype(jnp.bfloat16), dw_ref[0].astype(jnp.bfloat16),
                preferred_element_type=jnp.float32)
    out_ref[...] += coeff * y


def kernel(x, weights, expert_indices, gate_up, down, per_expert_scale):
    B, L, D = x.shape
    K = weights.shape[-1]
    E, _, F2 = gate_up.shape
    F = F2 // 2
    N = B * L

    x_flat = x.reshape(N, D)
    w_flat = weights.reshape(N, K)
    idx_flat = expert_indices.reshape(N, K)

    out = pl.pallas_call(
        functools.partial(_moe_kernel, F=F),
        grid=(E,),
        in_specs=[
            pl.BlockSpec((N, K), lambda e: (0, 0)),
            pl.BlockSpec((N, K), lambda e: (0, 0)),
            pl.BlockSpec((N, D), lambda e: (0, 0)),
            pl.BlockSpec(memory_space=pltpu.SMEM),
            pl.BlockSpec((1, D, F2), lambda e: (e, 0, 0)),
            pl.BlockSpec((1, F, D), lambda e: (e, 0, 0)),
        ],
        out_specs=pl.BlockSpec((N, D), lambda e: (0, 0)),
        out_shape=jax.ShapeDtypeStruct((N, D), jnp.float32),
    )(idx_flat, w_flat, x_flat, per_expert_scale, gate_up, down)

    return out.reshape(B, L, D)
